# baseline (speedup 1.0000x reference)
import functools

import jax
import jax.numpy as jnp
from jax import lax
from jax.experimental import pallas as pl
from jax.experimental.pallas import tpu as pltpu
from jax.experimental.pallas import tpu_sc as plsc

_VOCAB = 1000000
_D = 32
_NC = 2
_NS = 16
_NW = _NC * _NS
_BB = 128


@functools.lru_cache(maxsize=None)
def _build(batch: int, hist: int):
    nv = batch // _BB
    n_units = hist * nv
    upw = n_units // _NW
    nbuf = 4
    nt = upw // nbuf
    assert upw * _NW == n_units and nt * nbuf == upw
    mesh = plsc.VectorSubcoreMesh(
        core_axis_name="c", subcore_axis_name="s", num_cores=_NC, num_subcores=_NS
    )

    @functools.partial(
        pl.kernel,
        out_type=jax.ShapeDtypeStruct((hist, _D // 8, nv, 8 * _BB), jnp.float32),
        mesh=mesh,
        compiler_params=pltpu.CompilerParams(
            use_tc_tiling_on_sc=False, needs_layout_passes=False
        ),
        scratch_types=[
            pltpu.VMEM((upw * _BB,), jnp.int32),
            *([pltpu.VMEM((_BB, _D), jnp.float32)] * 4),
            *([pltpu.VMEM((_D * _BB,), jnp.float32)] * 4),
            *([pltpu.SemaphoreType.DMA] * 8),
        ],
    )
    def gather_kernel(table, idx, out,
                      idx_all, gat_v0, gat_v1, gat_v2, gat_v3,
                      tr_v0, tr_v1, tr_v2, tr_v3,
                      g0, g1, g2, g3, s0, s1, s2, s3):
        gat_v = (gat_v0, gat_v1, gat_v2, gat_v3)
        tr_v = (tr_v0, tr_v1, tr_v2, tr_v3)
        gsem = (g0, g1, g2, g3)
        ssem = (s0, s1, s2, s3)
        wid = lax.axis_index("s") * _NC + lax.axis_index("c")
        u0 = wid * upw
        row16 = jax.lax.iota(jnp.int32, 16)
        cpat = [lax.rem(row16 + k, 16) for k in range(16)]
        wpat = [cpat[k] * _BB + row16 for k in range(16)]

        def fire(u, b):
            lu = u - u0
            pltpu.async_copy(
                table.at[idx_all.at[pl.ds(lu * _BB, _BB)]], gat_v[b], gsem[b]
            )

        def process(u, b):
            h = u // nv
            v = lax.rem(u, nv)
            pltpu.make_async_copy(
                table.at[pl.ds(0, _BB)], gat_v[b], gsem[b]
            ).wait()
            @plsc.parallel_loop(0, (_BB // 16) * (_D // 16), unroll=8)
            def _transpose(bi):
                r0 = lax.rem(bi, _BB // 16) * 16
                c0 = (bi // (_BB // 16)) * 16
                rvec = row16 + r0
                off = c0 * _BB + r0
                for k in range(16):
                    val = plsc.load_gather(gat_v[b], [rvec, cpat[k] + c0])
                    plsc.store_scatter(tr_v[b], [wpat[k] + off], val)
            for tt in range(_D // 8):
                pltpu.async_copy(
                    tr_v[b].at[pl.ds(1024 * tt, 1024)], out.at[h, tt, v], ssem[b]
                )

        def drain_stores(b):
            pltpu.make_async_copy(
                table.at[pl.ds(0, _BB)], gat_v[b], ssem[b]
            ).wait()

        pltpu.sync_copy(idx.at[pl.ds(u0 * _BB, upw * _BB)], idx_all)

        for b in range(nbuf):
            fire(u0 + b, b)

        @pl.loop(0, nt)
        def _steady(t):
            for b in range(nbuf):
                u = u0 + nbuf * t + b

                @pl.when(t >= 1)
                def _():
                    drain_stores(b)

                process(u, b)

                @pl.when(t < nt - 1)
                def _():
                    fire(u + nbuf, b)

        for b in range(nbuf):
            drain_stores(b)

    return gather_kernel


def kernel(src, table):
    batch, hist = src.shape
    srcf = src.T.reshape(-1).astype(jnp.int32)
    x = _build(batch, hist)(table, srcf)
    x = x.reshape(hist, _D // 8, batch // _BB, 8, _BB)
    return x.transpose((2, 4, 0, 1, 3)).reshape(batch, hist, _D)


# speedup vs baseline: 1.0139x; 1.0139x over previous
import functools

import jax
import jax.numpy as jnp
from jax import lax
from jax.experimental import pallas as pl
from jax.experimental.pallas import tpu as pltpu
from jax.experimental.pallas import tpu_sc as plsc

_VOCAB = 1000000
_D = 32
_NC = 2
_NS = 16
_NW = _NC * _NS
_BB = 128


@functools.lru_cache(maxsize=None)
def _build(batch: int, hist: int):
    nv = batch // _BB
    n_units = hist * nv
    upw = n_units // _NW
    nbuf = 4
    nt = upw // nbuf
    assert upw * _NW == n_units and nt * nbuf == upw
    mesh = plsc.VectorSubcoreMesh(
        core_axis_name="c", subcore_axis_name="s", num_cores=_NC, num_subcores=_NS
    )

    @functools.partial(
        pl.kernel,
        out_type=jax.ShapeDtypeStruct((hist, _D // 8, nv, 8 * _BB), jnp.float32),
        mesh=mesh,
        compiler_params=pltpu.CompilerParams(
            use_tc_tiling_on_sc=False, needs_layout_passes=False
        ),
        scratch_types=[
            pltpu.VMEM((upw * _BB,), jnp.int32),
            *([pltpu.VMEM((_BB, _D), jnp.float32)] * 4),
            *([pltpu.VMEM((_D * _BB,), jnp.float32)] * 4),
            *([pltpu.SemaphoreType.DMA] * 8),
        ],
    )
    def gather_kernel(table, idx, out,
                      idx_all, gat_v0, gat_v1, gat_v2, gat_v3,
                      tr_v0, tr_v1, tr_v2, tr_v3,
                      g0, g1, g2, g3, s0, s1, s2, s3):
        gat_v = (gat_v0, gat_v1, gat_v2, gat_v3)
        tr_v = (tr_v0, tr_v1, tr_v2, tr_v3)
        gsem = (g0, g1, g2, g3)
        ssem = (s0, s1, s2, s3)
        wid = lax.axis_index("s") * _NC + lax.axis_index("c")
        u0 = wid * upw
        row16 = jax.lax.iota(jnp.int32, 16)
        cpat = [lax.rem(row16 + k, 16) for k in range(16)]
        wpat = [cpat[k] * _BB + row16 for k in range(16)]

        def fire(u, b):
            lu = u - u0
            pltpu.async_copy(
                table.at[idx_all.at[pl.ds(lu * _BB, _BB)]], gat_v[b], gsem[b]
            )

        def process(u, b):
            h = u // nv
            v = lax.rem(u, nv)
            pltpu.make_async_copy(
                table.at[pl.ds(0, _BB)], gat_v[b], gsem[b]
            ).wait()
            @plsc.parallel_loop(0, (_BB // 16) * (_D // 16), unroll=4)
            def _transpose(bi):
                r0 = lax.rem(bi, _BB // 16) * 16
                c0 = (bi // (_BB // 16)) * 16
                rvec = row16 + r0
                off = c0 * _BB + r0
                for k in range(16):
                    val = plsc.load_gather(gat_v[b], [rvec, cpat[k] + c0])
                    plsc.store_scatter(tr_v[b], [wpat[k] + off], val)
            for tt in range(_D // 8):
                pltpu.async_copy(
                    tr_v[b].at[pl.ds(1024 * tt, 1024)], out.at[h, tt, v], ssem[b]
                )

        def drain_stores(b):
            pltpu.make_async_copy(
                table.at[pl.ds(0, _BB)], gat_v[b], ssem[b]
            ).wait()

        pltpu.sync_copy(idx.at[pl.ds(u0 * _BB, upw * _BB)], idx_all)

        for b in range(nbuf):
            fire(u0 + b, b)

        @pl.loop(0, nt)
        def _steady(t):
            for b in range(nbuf):
                u = u0 + nbuf * t + b

                @pl.when(t >= 1)
                def _():
                    drain_stores(b)

                process(u, b)

                @pl.when(t < nt - 1)
                def _():
                    fire(u + nbuf, b)

        for b in range(nbuf):
            drain_stores(b)

    return gather_kernel


def kernel(src, table):
    batch, hist = src.shape
    srcf = src.T.reshape(-1).astype(jnp.int32)
    x = _build(batch, hist)(table, srcf)
    x = x.reshape(hist, _D // 8, batch // _BB, 8, _BB)
    return x.transpose((2, 4, 0, 1, 3)).reshape(batch, hist, _D)
